# baseline (speedup 1.0000x reference)
import functools

import jax
import jax.numpy as jnp
from jax import lax
from jax.experimental import pallas as pl
from jax.experimental.pallas import tpu as pltpu
from jax.experimental.pallas import tpu_sc as plsc

N_PAPER = 10000
N_AUTHOR = 10000
D = 256
H = 256
C = 64
E = 160000
ALPHA = 0.01

NC = 2
NS = 16
NW = NC * NS
CHUNK = 128
NBUF = 4
DUMP = N_PAPER
NROWS_PAD = 10112
ZSTRIPE = NROWS_PAD // NS

_f32 = jnp.float32


_DENSE_BLK = 1000


def _dense_body(x_ref, w_ref, b_ref, wout_ref, o_ref):
    h = jnp.dot(x_ref[...], w_ref[...], preferred_element_type=_f32)
    h = jnp.maximum(h + b_ref[...], 0.0)
    o_ref[...] = jnp.dot(h, wout_ref[...], preferred_element_type=_f32)


def _dense_project(x, w, b, w_out):
    n = x.shape[0]
    return pl.pallas_call(
        _dense_body,
        grid=(n // _DENSE_BLK,),
        in_specs=[
            pl.BlockSpec((_DENSE_BLK, D), lambda i: (i, 0)),
            pl.BlockSpec((D, H), lambda i: (0, 0)),
            pl.BlockSpec((1, H), lambda i: (0, 0)),
            pl.BlockSpec((H, C), lambda i: (0, 0)),
        ],
        out_specs=pl.BlockSpec((_DENSE_BLK, C), lambda i: (i, 0)),
        out_shape=jax.ShapeDtypeStruct((n, C), _f32),
    )(x, w, b.reshape(1, H), w_out)


def _segsum_body(ks, tids, *refs):
    np_ = len(ks)
    nt = max(tids) + 1
    it = iter(refs)
    tables = [next(it) for _ in range(nt)]
    srcs = [next(it) for _ in range(np_)]
    dsts = [next(it) for _ in range(np_)]
    zeros = next(it)
    out = next(it)
    srcvs = [next(it) for _ in range(np_)]
    dstvs = [next(it) for _ in range(np_)]
    rows = next(it)
    acc = next(it)
    gsems = [next(it) for _ in range(NBUF)]
    ssems = [next(it) for _ in range(NBUF)]

    c = lax.axis_index("c")
    s = lax.axis_index("s")
    wid = c * NS + s
    pltpu.sync_copy(zeros.at[pl.ds(s * ZSTRIPE, ZSTRIPE)],
                    acc.at[pl.ds(s * ZSTRIPE, ZSTRIPE)])
    for pi in range(np_):
        pltpu.sync_copy(srcs[pi].at[wid], srcvs[pi])
        pltpu.sync_copy(dsts[pi].at[wid], dstvs[pi])
    plsc.subcore_barrier()

    for pi in range(np_):
        table, srcv, dstv, k = tables[tids[pi]], srcvs[pi], dstvs[pi], ks[pi]
        for b in range(NBUF):
            pltpu.async_copy(table.at[srcv.at[b]], rows.at[b], gsems[b])

        def group(i, carry, table=table, srcv=srcv, dstv=dstv, k=k):
            for b in range(NBUF):
                j = i * NBUF + b
                pltpu.make_async_copy(table.at[srcv.at[j]], rows.at[b],
                                      gsems[b]).wait()
                pltpu.async_copy(rows.at[b], acc.at[dstv.at[j]], ssems[b],
                                 add=True)
            for b in range(NBUF):
                j = i * NBUF + b
                pltpu.make_async_copy(rows.at[b], acc.at[dstv.at[j]],
                                      ssems[b]).wait()

                @pl.when(j + NBUF < k)
                def _(b=b, j=j):
                    pltpu.async_copy(table.at[srcv.at[j + NBUF]], rows.at[b],
                                     gsems[b])

            return carry

        lax.fori_loop(0, k // NBUF, group, 0)

    plsc.subcore_barrier()
    pltpu.sync_copy(acc.at[pl.ds(s * ZSTRIPE, ZSTRIPE)],
                    out.at[c, pl.ds(s * ZSTRIPE, ZSTRIPE)])


def _sc_segsum(tables, srcs, dsts, tids, zeros):
    ks = [s.shape[1] for s in srcs]
    mesh = plsc.VectorSubcoreMesh(core_axis_name="c", subcore_axis_name="s")
    scratch = ([pltpu.VMEM((k, CHUNK), jnp.int32) for k in ks] * 2
               + [pltpu.VMEM((NBUF, CHUNK, C), _f32),
                  pltpu.VMEM_SHARED((NROWS_PAD, C), _f32)]
               + [pltpu.SemaphoreType.DMA] * (2 * NBUF))
    kern = functools.partial(
        pl.kernel,
        out_type=jax.ShapeDtypeStruct((NC, NROWS_PAD, C), _f32),
        mesh=mesh,
        scratch_types=scratch,
        compiler_params=pltpu.CompilerParams(use_tc_tiling_on_sc=False),
    )(functools.partial(_segsum_body, ks, tids))
    return kern(*tables, *srcs, *dsts, zeros)


_CBLK = 1000


def _add2_body(p_ref, o_ref):
    o_ref[...] = p_ref[0] + p_ref[1]


def _add_partials(parts):
    return pl.pallas_call(
        _add2_body,
        grid=(N_PAPER // _CBLK,),
        in_specs=[pl.BlockSpec((NC, _CBLK, C), lambda i: (0, i, 0))],
        out_specs=pl.BlockSpec((_CBLK, C), lambda i: (i, 0)),
        out_shape=jax.ShapeDtypeStruct((N_PAPER, C), _f32),
    )(parts)


def _final_body(p_ref, mw_ref, q_ref, b_ref, o_ref):
    o_ref[...] = ((ALPHA * ALPHA) * p_ref[...] + (2.0 * ALPHA) * mw_ref[...]
                  + q_ref[0] + q_ref[1] + b_ref[0])


def _final_combine(p, mw, q_parts, b_out):
    return pl.pallas_call(
        _final_body,
        grid=(N_PAPER // _CBLK,),
        in_specs=[
            pl.BlockSpec((_CBLK, C), lambda i: (i, 0)),
            pl.BlockSpec((_CBLK, C), lambda i: (i, 0)),
            pl.BlockSpec((NC, _CBLK, C), lambda i: (0, i, 0)),
            pl.BlockSpec((1, C), lambda i: (0, 0)),
        ],
        out_specs=pl.BlockSpec((_CBLK, C), lambda i: (i, 0)),
        out_shape=jax.ShapeDtypeStruct((N_PAPER, C), _f32),
    )(p, mw, q_parts, b_out.reshape(1, C))


def _pad_edges(src, dst):
    n = src.shape[0]
    per = NBUF * NW * CHUNK
    n_pad = -(-n // per) * per
    src = jnp.pad(src, (0, n_pad - n))
    dst = jnp.pad(dst, (0, n_pad - n), constant_values=DUMP)
    k = n_pad // (NW * CHUNK)
    return src.reshape(NW, k, CHUNK), dst.reshape(NW, k, CHUNK)


def kernel(x_paper, x_author, edge_index_writes, edge_index_cites,
           W_paper, b_paper, W_author, b_author, W_out, b_out):
    p = _dense_project(x_paper, W_paper, b_paper, W_out)
    a = _dense_project(x_author, W_author, b_author, W_out)

    ws, wd = (edge_index_writes[0].astype(jnp.int32),
              edge_index_writes[1].astype(jnp.int32))
    cs, cd = (edge_index_cites[0].astype(jnp.int32),
              edge_index_cites[1].astype(jnp.int32))

    zeros = jnp.zeros((NROWS_PAD, C), _f32)

    srcw3, dstw3 = _pad_edges(ws, wd)
    srcc3, dstc3 = _pad_edges(cs, cd)

    mw_parts = _sc_segsum([p, a], [srcw3, srcc3], [dstw3, dstc3],
                          tids=[1, 0], zeros=zeros)
    mw = _add_partials(mw_parts)

    q_parts = _sc_segsum([mw], [srcc3], [dstc3], tids=[0], zeros=zeros)

    return _final_combine(p, mw, q_parts, b_out)


# speedup vs baseline: 1.0912x; 1.0912x over previous
import functools

import jax
import jax.numpy as jnp
from jax import lax
from jax.experimental import pallas as pl
from jax.experimental.pallas import tpu as pltpu
from jax.experimental.pallas import tpu_sc as plsc

N_PAPER = 10000
N_AUTHOR = 10000
D = 256
H = 256
C = 64
E = 160000
ALPHA = 0.01

NC = 2
NS = 16
NW = NC * NS
CHUNK = 128
NBUF = 4
DUMP = N_PAPER
NROWS_PAD = 10112
ZSTRIPE = NROWS_PAD // NS

_f32 = jnp.float32


_DENSE_BLK = 1000


def _dense_body(x_ref, w_ref, b_ref, wout_ref, o_ref):
    h = jnp.dot(x_ref[...], w_ref[...], preferred_element_type=_f32)
    h = jnp.maximum(h + b_ref[...], 0.0)
    o_ref[...] = jnp.dot(h, wout_ref[...], preferred_element_type=_f32)


def _dense_project(x, w, b, w_out):
    n = x.shape[0]
    return pl.pallas_call(
        _dense_body,
        grid=(n // _DENSE_BLK,),
        in_specs=[
            pl.BlockSpec((_DENSE_BLK, D), lambda i: (i, 0)),
            pl.BlockSpec((D, H), lambda i: (0, 0)),
            pl.BlockSpec((1, H), lambda i: (0, 0)),
            pl.BlockSpec((H, C), lambda i: (0, 0)),
        ],
        out_specs=pl.BlockSpec((_DENSE_BLK, C), lambda i: (i, 0)),
        out_shape=jax.ShapeDtypeStruct((n, C), _f32),
    )(x, w, b.reshape(1, H), w_out)


def _segsum_body(ks, tids, *refs):
    np_ = len(ks)
    nt = max(tids) + 1
    it = iter(refs)
    tables = [next(it) for _ in range(nt)]
    srcs = [next(it) for _ in range(np_)]
    dsts = [next(it) for _ in range(np_)]
    zeros = next(it)
    out = next(it)
    srcvs = [next(it) for _ in range(np_)]
    dstvs = [next(it) for _ in range(np_)]
    rows = next(it)
    acc = next(it)
    gsems = [next(it) for _ in range(NBUF)]
    ssems = [next(it) for _ in range(NBUF)]

    c = lax.axis_index("c")
    s = lax.axis_index("s")
    wid = c * NS + s
    pltpu.sync_copy(zeros.at[pl.ds(s * ZSTRIPE, ZSTRIPE)],
                    acc.at[pl.ds(s * ZSTRIPE, ZSTRIPE)])
    for pi in range(np_):
        pltpu.sync_copy(srcs[pi].at[wid], srcvs[pi])
        pltpu.sync_copy(dsts[pi].at[wid], dstvs[pi])
    plsc.subcore_barrier()

    for pi in range(np_):
        table, srcv, dstv, k = tables[tids[pi]], srcvs[pi], dstvs[pi], ks[pi]
        for b in range(NBUF):
            pltpu.async_copy(table.at[srcv.at[b]], rows.at[b], gsems[b])

        def group(i, carry, table=table, srcv=srcv, dstv=dstv, k=k):
            for b in range(NBUF):
                j = i * NBUF + b
                pltpu.make_async_copy(table.at[srcv.at[j]], rows.at[b],
                                      gsems[b]).wait()
                pltpu.async_copy(rows.at[b], acc.at[dstv.at[j]], ssems[b],
                                 add=True)
            for b in range(NBUF):
                j = i * NBUF + b
                pltpu.make_async_copy(rows.at[b], acc.at[dstv.at[j]],
                                      ssems[b]).wait()

                @pl.when(j + NBUF < k)
                def _(b=b, j=j):
                    pltpu.async_copy(table.at[srcv.at[j + NBUF]], rows.at[b],
                                     gsems[b])

            return carry

        lax.fori_loop(0, k // NBUF, group, 0)

    plsc.subcore_barrier()
    pltpu.sync_copy(acc.at[pl.ds(s * ZSTRIPE, ZSTRIPE)],
                    out.at[c, pl.ds(s * ZSTRIPE, ZSTRIPE)])


def _sc_segsum(tables, srcs, dsts, tids, zeros):
    ks = [s.shape[1] for s in srcs]
    mesh = plsc.VectorSubcoreMesh(core_axis_name="c", subcore_axis_name="s")
    scratch = ([pltpu.VMEM((k, CHUNK), jnp.int32) for k in ks] * 2
               + [pltpu.VMEM((NBUF, CHUNK, C), _f32),
                  pltpu.VMEM_SHARED((NROWS_PAD, C), _f32)]
               + [pltpu.SemaphoreType.DMA] * (2 * NBUF))
    kern = functools.partial(
        pl.kernel,
        out_type=jax.ShapeDtypeStruct((NC, NROWS_PAD, C), _f32),
        mesh=mesh,
        scratch_types=scratch,
        compiler_params=pltpu.CompilerParams(use_tc_tiling_on_sc=False),
    )(functools.partial(_segsum_body, ks, tids))
    return kern(*tables, *srcs, *dsts, zeros)


_CBLK = 1000


def _add2_body(p_ref, o_ref):
    o_ref[...] = p_ref[0] + p_ref[1]


def _add_partials(parts):
    return pl.pallas_call(
        _add2_body,
        grid=(N_PAPER // _CBLK,),
        in_specs=[pl.BlockSpec((NC, _CBLK, C), lambda i: (0, i, 0))],
        out_specs=pl.BlockSpec((_CBLK, C), lambda i: (i, 0)),
        out_shape=jax.ShapeDtypeStruct((N_PAPER, C), _f32),
    )(parts)


def _final_body(p_ref, mw_ref, q_ref, b_ref, o_ref):
    o_ref[...] = ((ALPHA * ALPHA) * p_ref[...] + (2.0 * ALPHA) * mw_ref[...]
                  + q_ref[0] + q_ref[1] + b_ref[0])


def _final_combine(p, mw, q_parts, b_out):
    return pl.pallas_call(
        _final_body,
        grid=(N_PAPER // _CBLK,),
        in_specs=[
            pl.BlockSpec((_CBLK, C), lambda i: (i, 0)),
            pl.BlockSpec((_CBLK, C), lambda i: (i, 0)),
            pl.BlockSpec((NC, _CBLK, C), lambda i: (0, i, 0)),
            pl.BlockSpec((1, C), lambda i: (0, 0)),
        ],
        out_specs=pl.BlockSpec((_CBLK, C), lambda i: (i, 0)),
        out_shape=jax.ShapeDtypeStruct((N_PAPER, C), _f32),
    )(p, mw, q_parts, b_out.reshape(1, C))


def _pad_edges(src, dst):
    n = src.shape[0]
    per = NBUF * NW * CHUNK
    n_pad = -(-n // per) * per
    src = jnp.pad(src, (0, n_pad - n))
    pad_dst = DUMP + (jnp.arange(n_pad - n, dtype=jnp.int32)
                      % (NROWS_PAD - N_PAPER))
    dst = jnp.concatenate([dst, pad_dst])
    k = n_pad // (NW * CHUNK)
    return src.reshape(NW, k, CHUNK), dst.reshape(NW, k, CHUNK)


def kernel(x_paper, x_author, edge_index_writes, edge_index_cites,
           W_paper, b_paper, W_author, b_author, W_out, b_out):
    p = _dense_project(x_paper, W_paper, b_paper, W_out)
    a = _dense_project(x_author, W_author, b_author, W_out)

    ws, wd = (edge_index_writes[0].astype(jnp.int32),
              edge_index_writes[1].astype(jnp.int32))
    cs, cd = (edge_index_cites[0].astype(jnp.int32),
              edge_index_cites[1].astype(jnp.int32))

    zeros = jnp.zeros((NROWS_PAD, C), _f32)

    srcw3, dstw3 = _pad_edges(ws, wd)
    srcc3, dstc3 = _pad_edges(cs, cd)

    mw_parts = _sc_segsum([p, a], [srcw3, srcc3], [dstw3, dstc3],
                          tids=[1, 0], zeros=zeros)
    mw = _add_partials(mw_parts)

    q_parts = _sc_segsum([mw], [srcc3], [dstc3], tids=[0], zeros=zeros)

    return _final_combine(p, mw, q_parts, b_out)


# speedup vs baseline: 2.0312x; 1.8615x over previous
import functools

import jax
import jax.numpy as jnp
from jax import lax
from jax.experimental import pallas as pl
from jax.experimental.pallas import tpu as pltpu
from jax.experimental.pallas import tpu_sc as plsc

N_PAPER = 10000
N_AUTHOR = 10000
D = 256
H = 256
C = 64
E = 160000
ALPHA = 0.01

NC = 2
NS = 16
NW = NC * NS
CHUNK = 128
NBUF = 4
DUMP = N_PAPER
NROWS_PAD = 10112
ZSTRIPE = NROWS_PAD // NS

_f32 = jnp.float32


_DENSE_BLK = 1000


def _dense_body(x_ref, w_ref, b_ref, wout_ref, o_ref):
    h = jnp.dot(x_ref[...], w_ref[...], preferred_element_type=_f32)
    h = jnp.maximum(h + b_ref[...], 0.0)
    o_ref[...] = jnp.dot(h, wout_ref[...], preferred_element_type=_f32)


def _dense_project(x, w, b, w_out):
    n = x.shape[0]
    return pl.pallas_call(
        _dense_body,
        grid=(n // _DENSE_BLK,),
        in_specs=[
            pl.BlockSpec((_DENSE_BLK, D), lambda i: (i, 0)),
            pl.BlockSpec((D, H), lambda i: (0, 0)),
            pl.BlockSpec((1, H), lambda i: (0, 0)),
            pl.BlockSpec((H, C), lambda i: (0, 0)),
        ],
        out_specs=pl.BlockSpec((_DENSE_BLK, C), lambda i: (i, 0)),
        out_shape=jax.ShapeDtypeStruct((NROWS_PAD, C), _f32),
    )(x, w, b.reshape(1, H), w_out)


def _segsum_body(ks, *refs):
    np_ = len(ks)
    it = iter(refs)
    table_hbm = next(it)
    srcs = [next(it) for _ in range(np_)]
    dsts = [next(it) for _ in range(np_)]
    zeros = next(it)
    out = next(it)
    srcvs = [next(it) for _ in range(np_)]
    dstvs = [next(it) for _ in range(np_)]
    rows = next(it)
    acc = next(it)
    tbl = next(it)
    gsems = [next(it) for _ in range(NBUF)]
    ssems = [next(it) for _ in range(NBUF)]

    c = lax.axis_index("c")
    s = lax.axis_index("s")
    wid = c * NS + s
    stripe = pl.ds(s * ZSTRIPE, ZSTRIPE)
    pltpu.sync_copy(zeros.at[stripe], acc.at[stripe])
    pltpu.sync_copy(table_hbm.at[stripe], tbl.at[stripe])
    for pi in range(np_):
        pltpu.sync_copy(srcs[pi].at[wid], srcvs[pi])
        pltpu.sync_copy(dsts[pi].at[wid], dstvs[pi])
    plsc.subcore_barrier()

    for pi in range(np_):
        table, srcv, dstv, k = tbl, srcvs[pi], dstvs[pi], ks[pi]
        for b in range(NBUF):
            pltpu.async_copy(table.at[srcv.at[b]], rows.at[b], gsems[b])

        def group(i, carry, table=table, srcv=srcv, dstv=dstv, k=k):
            for b in range(NBUF):
                j = i * NBUF + b
                pltpu.make_async_copy(table.at[srcv.at[j]], rows.at[b],
                                      gsems[b]).wait()
                pltpu.async_copy(rows.at[b], acc.at[dstv.at[j]], ssems[b],
                                 add=True)
            for b in range(NBUF):
                j = i * NBUF + b
                pltpu.make_async_copy(rows.at[b], acc.at[dstv.at[j]],
                                      ssems[b]).wait()

                @pl.when(j + NBUF < k)
                def _(b=b, j=j):
                    pltpu.async_copy(table.at[srcv.at[j + NBUF]], rows.at[b],
                                     gsems[b])

            return carry

        lax.fori_loop(0, k // NBUF, group, 0)

    plsc.subcore_barrier()
    pltpu.sync_copy(acc.at[pl.ds(s * ZSTRIPE, ZSTRIPE)],
                    out.at[c, pl.ds(s * ZSTRIPE, ZSTRIPE)])


def _sc_segsum(table, srcs, dsts, zeros):
    ks = [s.shape[1] for s in srcs]
    mesh = plsc.VectorSubcoreMesh(core_axis_name="c", subcore_axis_name="s")
    scratch = ([pltpu.VMEM((k, CHUNK), jnp.int32) for k in ks] * 2
               + [pltpu.VMEM((NBUF, CHUNK, C), _f32),
                  pltpu.VMEM_SHARED((NROWS_PAD, C), _f32),
                  pltpu.VMEM_SHARED((NROWS_PAD, C), _f32)]
               + [pltpu.SemaphoreType.DMA] * (2 * NBUF))
    kern = functools.partial(
        pl.kernel,
        out_type=jax.ShapeDtypeStruct((NC, NROWS_PAD, C), _f32),
        mesh=mesh,
        scratch_types=scratch,
        compiler_params=pltpu.CompilerParams(use_tc_tiling_on_sc=False),
    )(functools.partial(_segsum_body, ks))
    return kern(table, *srcs, *dsts, zeros)


_CBLK = 1000


def _add4_body(pa_ref, pb_ref, o_ref):
    o_ref[...] = (pa_ref[0] + pa_ref[1]) + (pb_ref[0] + pb_ref[1])


def _add_partials(parts_a, parts_b):
    return pl.pallas_call(
        _add4_body,
        grid=(N_PAPER // _CBLK,),
        in_specs=[pl.BlockSpec((NC, _CBLK, C), lambda i: (0, i, 0)),
                  pl.BlockSpec((NC, _CBLK, C), lambda i: (0, i, 0))],
        out_specs=pl.BlockSpec((_CBLK, C), lambda i: (i, 0)),
        out_shape=jax.ShapeDtypeStruct((NROWS_PAD, C), _f32),
    )(parts_a, parts_b)


def _final_body(p_ref, mw_ref, q_ref, b_ref, o_ref):
    o_ref[...] = ((ALPHA * ALPHA) * p_ref[...] + (2.0 * ALPHA) * mw_ref[...]
                  + q_ref[0] + q_ref[1] + b_ref[0])


def _final_combine(p, mw, q_parts, b_out):
    return pl.pallas_call(
        _final_body,
        grid=(N_PAPER // _CBLK,),
        in_specs=[
            pl.BlockSpec((_CBLK, C), lambda i: (i, 0)),
            pl.BlockSpec((_CBLK, C), lambda i: (i, 0)),
            pl.BlockSpec((NC, _CBLK, C), lambda i: (0, i, 0)),
            pl.BlockSpec((1, C), lambda i: (0, 0)),
        ],
        out_specs=pl.BlockSpec((_CBLK, C), lambda i: (i, 0)),
        out_shape=jax.ShapeDtypeStruct((N_PAPER, C), _f32),
    )(p, mw, q_parts, b_out.reshape(1, C))


def _pad_edges(src, dst):
    n = src.shape[0]
    per = NBUF * NW * CHUNK
    n_pad = -(-n // per) * per
    src = jnp.pad(src, (0, n_pad - n))
    pad_dst = DUMP + (jnp.arange(n_pad - n, dtype=jnp.int32)
                      % (NROWS_PAD - N_PAPER))
    dst = jnp.concatenate([dst, pad_dst])
    k = n_pad // (NW * CHUNK)
    return src.reshape(NW, k, CHUNK), dst.reshape(NW, k, CHUNK)


def kernel(x_paper, x_author, edge_index_writes, edge_index_cites,
           W_paper, b_paper, W_author, b_author, W_out, b_out):
    p = _dense_project(x_paper, W_paper, b_paper, W_out)
    a = _dense_project(x_author, W_author, b_author, W_out)

    ws, wd = (edge_index_writes[0].astype(jnp.int32),
              edge_index_writes[1].astype(jnp.int32))
    cs, cd = (edge_index_cites[0].astype(jnp.int32),
              edge_index_cites[1].astype(jnp.int32))

    zeros = jnp.zeros((NROWS_PAD, C), _f32)

    srcw3, dstw3 = _pad_edges(ws, wd)
    srcc3, dstc3 = _pad_edges(cs, cd)

    w_parts = _sc_segsum(a, [srcw3], [dstw3], zeros)
    c_parts = _sc_segsum(p, [srcc3], [dstc3], zeros)
    mw = _add_partials(w_parts, c_parts)

    q_parts = _sc_segsum(mw, [srcc3], [dstc3], zeros)

    return _final_combine(p, mw, q_parts, b_out)


# speedup vs baseline: 2.0962x; 1.0320x over previous
import functools

import jax
import jax.numpy as jnp
from jax import lax
from jax.experimental import pallas as pl
from jax.experimental.pallas import tpu as pltpu
from jax.experimental.pallas import tpu_sc as plsc

N_PAPER = 10000
N_AUTHOR = 10000
D = 256
H = 256
C = 64
E = 160000
ALPHA = 0.01

NC = 2
NS = 16
NW = NC * NS
CHUNK = 128
NBUF = 2
DUMP = N_PAPER
NROWS_PAD = 10112
ZSTRIPE = NROWS_PAD // NS
NQ = -(-ZSTRIPE // CHUNK)
NBUF2 = 2

_f32 = jnp.float32


_DENSE_BLK = 1000


def _dense_body(x_ref, w_ref, b_ref, wout_ref, o_ref):
    h = jnp.dot(x_ref[...], w_ref[...], preferred_element_type=_f32)
    h = jnp.maximum(h + b_ref[...], 0.0)
    o_ref[...] = jnp.dot(h, wout_ref[...], preferred_element_type=_f32)


def _dense_project(x, w, b, w_out):
    n = x.shape[0]
    return pl.pallas_call(
        _dense_body,
        grid=(n // _DENSE_BLK,),
        in_specs=[
            pl.BlockSpec((_DENSE_BLK, D), lambda i: (i, 0)),
            pl.BlockSpec((D, H), lambda i: (0, 0)),
            pl.BlockSpec((1, H), lambda i: (0, 0)),
            pl.BlockSpec((H, C), lambda i: (0, 0)),
        ],
        out_specs=pl.BlockSpec((_DENSE_BLK, C), lambda i: (i, 0)),
        out_shape=jax.ShapeDtypeStruct((NROWS_PAD, C), _f32),
    )(x, w, b.reshape(1, H), w_out)


def _edge_pipeline(table, srcv, dstv, k, rows, acc, gsems, ssems, nbuf):
    for b in range(nbuf):
        pltpu.async_copy(table.at[srcv.at[b]], rows.at[b], gsems[b])

    def group(i, carry):
        for b in range(nbuf):
            j = i * nbuf + b
            pltpu.make_async_copy(table.at[srcv.at[j]], rows.at[b],
                                  gsems[b]).wait()
            pltpu.async_copy(rows.at[b], acc.at[dstv.at[j]], ssems[b],
                             add=True)
        for b in range(nbuf):
            j = i * nbuf + b
            pltpu.make_async_copy(rows.at[b], acc.at[dstv.at[j]],
                                  ssems[b]).wait()

            @pl.when(j + nbuf < k)
            def _(b=b, j=j):
                pltpu.async_copy(table.at[srcv.at[j + nbuf]], rows.at[b],
                                 gsems[b])

        return carry

    lax.fori_loop(0, k // nbuf, group, 0)


def _segsum2_body(k, table_a, table_p, idx, zeros, out,
                  sv0, dv0, sv1, dv1, rows, acc, tbl, *sems):
    gsems, ssems = sems[:NBUF], sems[NBUF:]
    c = lax.axis_index("c")
    s = lax.axis_index("s")
    wid = c * NS + s
    stripe = pl.ds(s * ZSTRIPE, ZSTRIPE)
    pltpu.sync_copy(zeros.at[stripe], acc.at[stripe])
    pltpu.sync_copy(table_a.at[stripe], tbl.at[stripe])
    pltpu.sync_copy(idx.at[0, wid], sv0)
    pltpu.sync_copy(idx.at[1, wid], dv0)
    pltpu.sync_copy(idx.at[2, wid], sv1)
    pltpu.sync_copy(idx.at[3, wid], dv1)
    plsc.subcore_barrier()
    _edge_pipeline(tbl, sv0, dv0, k, rows, acc, gsems, ssems, NBUF)
    plsc.subcore_barrier()
    pltpu.sync_copy(table_p.at[stripe], tbl.at[stripe])
    plsc.subcore_barrier()
    _edge_pipeline(tbl, sv1, dv1, k, rows, acc, gsems, ssems, NBUF)
    plsc.subcore_barrier()
    pltpu.sync_copy(acc.at[stripe], out.at[c, stripe])


def _segsum_final_body(k, parts, idx, ident, zeros, out, mw_out,
                       sv, dv, idv, rows, acc, tbl, *sems):
    gsems, ssems = sems[:NBUF2], sems[NBUF2:]
    c = lax.axis_index("c")
    s = lax.axis_index("s")
    wid = c * NS + s
    stripe = pl.ds(s * ZSTRIPE, ZSTRIPE)
    pltpu.sync_copy(zeros.at[stripe], acc.at[stripe])
    pltpu.sync_copy(parts.at[0, stripe], tbl.at[stripe])
    pltpu.sync_copy(ident.at[s], idv)
    pltpu.sync_copy(idx.at[2, wid], sv)
    pltpu.sync_copy(idx.at[3, wid], dv)
    for q in range(NQ):
        b = q % NBUF2
        n_r = min(CHUNK, ZSTRIPE - q * CHUNK)
        pltpu.sync_copy(parts.at[1, pl.ds(s * ZSTRIPE + q * CHUNK, n_r)],
                        rows.at[b, pl.ds(0, n_r)])
        pltpu.sync_copy(rows.at[b], tbl.at[idv.at[q]], add=True)
    plsc.subcore_barrier()

    @pl.when(c == 0)
    def _():
        pltpu.sync_copy(tbl.at[stripe], mw_out.at[stripe])

    _edge_pipeline(tbl, sv, dv, k, rows, acc, gsems, ssems, NBUF2)
    plsc.subcore_barrier()
    pltpu.sync_copy(acc.at[stripe], out.at[c, stripe])


_MESH = dict(core_axis_name="c", subcore_axis_name="s")


def _sc_pass1(table_a, table_p, idx, zeros):
    k = idx.shape[2]
    scratch = ([pltpu.VMEM((k, CHUNK), jnp.int32)] * 4
               + [pltpu.VMEM((NBUF, CHUNK, C), _f32),
                  pltpu.VMEM_SHARED((NROWS_PAD, C), _f32),
                  pltpu.VMEM_SHARED((NROWS_PAD, C), _f32)]
               + [pltpu.SemaphoreType.DMA] * (2 * NBUF))
    kern = functools.partial(
        pl.kernel,
        out_type=jax.ShapeDtypeStruct((NC, NROWS_PAD, C), _f32),
        mesh=plsc.VectorSubcoreMesh(**_MESH),
        scratch_types=scratch,
        compiler_params=pltpu.CompilerParams(use_tc_tiling_on_sc=False),
    )(functools.partial(_segsum2_body, k))
    return kern(table_a, table_p, idx, zeros)


def _sc_pass2(parts, idx, ident, zeros):
    k = idx.shape[2]
    scratch = ([pltpu.VMEM((k, CHUNK), jnp.int32)] * 2
               + [pltpu.VMEM((NQ, CHUNK), jnp.int32),
                  pltpu.VMEM((NBUF2, CHUNK, C), _f32),
                  pltpu.VMEM_SHARED((NROWS_PAD, C), _f32),
                  pltpu.VMEM_SHARED((NROWS_PAD, C), _f32)]
               + [pltpu.SemaphoreType.DMA] * (2 * NBUF2))
    kern = functools.partial(
        pl.kernel,
        out_type=(jax.ShapeDtypeStruct((NC, NROWS_PAD, C), _f32),
                  jax.ShapeDtypeStruct((NROWS_PAD, C), _f32)),
        mesh=plsc.VectorSubcoreMesh(**_MESH),
        scratch_types=scratch,
        compiler_params=pltpu.CompilerParams(use_tc_tiling_on_sc=False),
    )(functools.partial(_segsum_final_body, k))
    return kern(parts, idx, ident, zeros)


_CBLK = 1000


def _final_body(p_ref, mw_ref, q_ref, b_ref, o_ref):
    o_ref[...] = ((ALPHA * ALPHA) * p_ref[...] + (2.0 * ALPHA) * mw_ref[...]
                  + q_ref[0] + q_ref[1] + b_ref[0])


def _final_combine(p, mw, q_parts, b_out):
    return pl.pallas_call(
        _final_body,
        grid=(N_PAPER // _CBLK,),
        in_specs=[
            pl.BlockSpec((_CBLK, C), lambda i: (i, 0)),
            pl.BlockSpec((_CBLK, C), lambda i: (i, 0)),
            pl.BlockSpec((NC, _CBLK, C), lambda i: (0, i, 0)),
            pl.BlockSpec((1, C), lambda i: (0, 0)),
        ],
        out_specs=pl.BlockSpec((_CBLK, C), lambda i: (i, 0)),
        out_shape=jax.ShapeDtypeStruct((N_PAPER, C), _f32),
    )(p, mw, q_parts, b_out.reshape(1, C))


def _pad_flat(v, fill_ramp):
    n = v.shape[0]
    per = NBUF * NW * CHUNK
    n_pad = -(-n // per) * per
    if fill_ramp:
        pad = DUMP + (jnp.arange(n_pad - n, dtype=jnp.int32)
                      % (NROWS_PAD - N_PAPER))
        return jnp.concatenate([v, pad])
    return jnp.pad(v, (0, n_pad - n))


def kernel(x_paper, x_author, edge_index_writes, edge_index_cites,
           W_paper, b_paper, W_author, b_author, W_out, b_out):
    p = _dense_project(x_paper, W_paper, b_paper, W_out)
    a = _dense_project(x_author, W_author, b_author, W_out)

    ei_w = edge_index_writes.astype(jnp.int32)
    ei_c = edge_index_cites.astype(jnp.int32)
    idx = jnp.stack([
        _pad_flat(ei_w[0], False), _pad_flat(ei_w[1], True),
        _pad_flat(ei_c[0], False), _pad_flat(ei_c[1], True),
    ])
    k = idx.shape[1] // (NW * CHUNK)
    idx = idx.reshape(4, NW, k, CHUNK)

    zeros = jnp.zeros((NROWS_PAD, C), _f32)
    s_col = jnp.arange(NS, dtype=jnp.int32)[:, None]
    x_row = jnp.arange(NQ * CHUNK, dtype=jnp.int32)[None, :]
    ident = jnp.where(
        x_row < ZSTRIPE, s_col * ZSTRIPE + x_row,
        N_PAPER + (s_col * (NQ * CHUNK - ZSTRIPE) + (x_row - ZSTRIPE))
        % (NROWS_PAD - N_PAPER)).reshape(NS, NQ, CHUNK)

    mw_parts = _sc_pass1(a, p, idx, zeros)
    q_parts, mw = _sc_pass2(mw_parts, idx, ident, zeros)
    return _final_combine(p, mw, q_parts, b_out)


# speedup vs baseline: 2.1552x; 1.0281x over previous
import functools

import jax
import jax.numpy as jnp
from jax import lax
from jax.experimental import pallas as pl
from jax.experimental.pallas import tpu as pltpu
from jax.experimental.pallas import tpu_sc as plsc

N_PAPER = 10000
N_AUTHOR = 10000
D = 256
H = 256
C = 64
E = 160000
ALPHA = 0.01

NC = 2
NS = 16
NW = NC * NS
CHUNK = 128
NBUF = 2
DUMP = N_PAPER
NROWS_PAD = 10112
ZSTRIPE = NROWS_PAD // NS
NQ = -(-ZSTRIPE // CHUNK)
NBUF2 = 2

_f32 = jnp.float32


_DENSE_BLK = 1000


def _dense_body(x_ref, w_ref, b_ref, wout_ref, o_ref):
    h = jnp.dot(x_ref[...], w_ref[...], preferred_element_type=_f32)
    h = jnp.maximum(h + b_ref[...], 0.0)
    o_ref[...] = jnp.dot(h, wout_ref[...], preferred_element_type=_f32)


def _dense_project(x, w, b, w_out):
    n = x.shape[0]
    return pl.pallas_call(
        _dense_body,
        grid=(n // _DENSE_BLK,),
        in_specs=[
            pl.BlockSpec((_DENSE_BLK, D), lambda i: (i, 0)),
            pl.BlockSpec((D, H), lambda i: (0, 0)),
            pl.BlockSpec((1, H), lambda i: (0, 0)),
            pl.BlockSpec((H, C), lambda i: (0, 0)),
        ],
        out_specs=pl.BlockSpec((_DENSE_BLK, C), lambda i: (i, 0)),
        out_shape=jax.ShapeDtypeStruct((NROWS_PAD, C), _f32),
    )(x, w, b.reshape(1, H), w_out)


def _edge_pipeline(table, srcv, dstv, k, rows, acc, gsems, ssems, nbuf):
    for b in range(nbuf):
        pltpu.async_copy(table.at[srcv.at[b]], rows.at[b], gsems[b])

    def group(i, carry):
        for b in range(nbuf):
            j = i * nbuf + b
            pltpu.make_async_copy(table.at[srcv.at[j]], rows.at[b],
                                  gsems[b]).wait()
            pltpu.async_copy(rows.at[b], acc.at[dstv.at[j]], ssems[b],
                             add=True)
        for b in range(nbuf):
            j = i * nbuf + b
            pltpu.make_async_copy(rows.at[b], acc.at[dstv.at[j]],
                                  ssems[b]).wait()

            @pl.when(j + nbuf < k)
            def _(b=b, j=j):
                pltpu.async_copy(table.at[srcv.at[j + nbuf]], rows.at[b],
                                 gsems[b])

        return carry

    lax.fori_loop(0, k // nbuf, group, 0)


def _segsum2_body(k, table_a, table_p, idx, zeros, out,
                  sv0, dv0, sv1, dv1, rows, acc, tbl, *sems):
    gsems, ssems = sems[:NBUF], sems[NBUF:]
    c = lax.axis_index("c")
    s = lax.axis_index("s")
    wid = c * NS + s
    stripe = pl.ds(s * ZSTRIPE, ZSTRIPE)
    stage = [(zeros.at[stripe], acc.at[stripe]),
             (table_a.at[stripe], tbl.at[stripe]),
             (idx.at[0, wid], sv0), (idx.at[1, wid], dv0),
             (idx.at[2, wid], sv1), (idx.at[3, wid], dv1)]
    for i, (src, dst) in enumerate(stage):
        pltpu.async_copy(src, dst, sems[i % (2 * NBUF)])
    for i, (src, dst) in enumerate(stage):
        pltpu.make_async_copy(src, dst, sems[i % (2 * NBUF)]).wait()
    plsc.subcore_barrier()
    _edge_pipeline(tbl, sv0, dv0, k, rows, acc, gsems, ssems, NBUF)
    plsc.subcore_barrier()
    pltpu.sync_copy(table_p.at[stripe], tbl.at[stripe])
    plsc.subcore_barrier()
    _edge_pipeline(tbl, sv1, dv1, k, rows, acc, gsems, ssems, NBUF)
    plsc.subcore_barrier()
    pltpu.sync_copy(acc.at[stripe], out.at[c, stripe])


def _segsum_final_body(k, parts, idx, ident, zeros, out, mw_out,
                       sv, dv, idv, rows, acc, tbl, *sems):
    gsems, ssems = sems[:NBUF2], sems[NBUF2:]
    c = lax.axis_index("c")
    s = lax.axis_index("s")
    wid = c * NS + s
    stripe = pl.ds(s * ZSTRIPE, ZSTRIPE)
    stage = [(zeros.at[stripe], acc.at[stripe]),
             (parts.at[0, stripe], tbl.at[stripe]),
             (ident.at[s], idv),
             (idx.at[2, wid], sv), (idx.at[3, wid], dv)]
    for i, (src, dst) in enumerate(stage):
        pltpu.async_copy(src, dst, sems[i % (2 * NBUF2)])
    for i, (src, dst) in enumerate(stage):
        pltpu.make_async_copy(src, dst, sems[i % (2 * NBUF2)]).wait()
    def bounce(q, b):
        n_r = min(CHUNK, ZSTRIPE - q * CHUNK)
        return (parts.at[1, pl.ds(s * ZSTRIPE + q * CHUNK, n_r)],
                rows.at[b, pl.ds(0, n_r)])

    pltpu.async_copy(*bounce(0, 0), gsems[0])
    for q in range(NQ):
        b = q % NBUF2
        pltpu.make_async_copy(*bounce(q, b), gsems[b]).wait()
        if q + 1 < NQ:
            pltpu.async_copy(*bounce(q + 1, (q + 1) % NBUF2),
                             gsems[(q + 1) % NBUF2])
        pltpu.sync_copy(rows.at[b], tbl.at[idv.at[q]], add=True)
    plsc.subcore_barrier()

    @pl.when(c == 0)
    def _():
        pltpu.sync_copy(tbl.at[stripe], mw_out.at[stripe])

    _edge_pipeline(tbl, sv, dv, k, rows, acc, gsems, ssems, NBUF2)
    plsc.subcore_barrier()
    pltpu.sync_copy(acc.at[stripe], out.at[c, stripe])


_MESH = dict(core_axis_name="c", subcore_axis_name="s")


def _sc_pass1(table_a, table_p, idx, zeros):
    k = idx.shape[2]
    scratch = ([pltpu.VMEM((k, CHUNK), jnp.int32)] * 4
               + [pltpu.VMEM((NBUF, CHUNK, C), _f32),
                  pltpu.VMEM_SHARED((NROWS_PAD, C), _f32),
                  pltpu.VMEM_SHARED((NROWS_PAD, C), _f32)]
               + [pltpu.SemaphoreType.DMA] * (2 * NBUF))
    kern = functools.partial(
        pl.kernel,
        out_type=jax.ShapeDtypeStruct((NC, NROWS_PAD, C), _f32),
        mesh=plsc.VectorSubcoreMesh(**_MESH),
        scratch_types=scratch,
        compiler_params=pltpu.CompilerParams(use_tc_tiling_on_sc=False),
    )(functools.partial(_segsum2_body, k))
    return kern(table_a, table_p, idx, zeros)


def _sc_pass2(parts, idx, ident, zeros):
    k = idx.shape[2]
    scratch = ([pltpu.VMEM((k, CHUNK), jnp.int32)] * 2
               + [pltpu.VMEM((NQ, CHUNK), jnp.int32),
                  pltpu.VMEM((NBUF2, CHUNK, C), _f32),
                  pltpu.VMEM_SHARED((NROWS_PAD, C), _f32),
                  pltpu.VMEM_SHARED((NROWS_PAD, C), _f32)]
               + [pltpu.SemaphoreType.DMA] * (2 * NBUF2))
    kern = functools.partial(
        pl.kernel,
        out_type=(jax.ShapeDtypeStruct((NC, NROWS_PAD, C), _f32),
                  jax.ShapeDtypeStruct((NROWS_PAD, C), _f32)),
        mesh=plsc.VectorSubcoreMesh(**_MESH),
        scratch_types=scratch,
        compiler_params=pltpu.CompilerParams(use_tc_tiling_on_sc=False),
    )(functools.partial(_segsum_final_body, k))
    return kern(parts, idx, ident, zeros)


_CBLK = 1000


def _final_body(p_ref, mw_ref, q_ref, b_ref, o_ref):
    o_ref[...] = ((ALPHA * ALPHA) * p_ref[...] + (2.0 * ALPHA) * mw_ref[...]
                  + q_ref[0] + q_ref[1] + b_ref[0])


def _final_combine(p, mw, q_parts, b_out):
    return pl.pallas_call(
        _final_body,
        grid=(N_PAPER // _CBLK,),
        in_specs=[
            pl.BlockSpec((_CBLK, C), lambda i: (i, 0)),
            pl.BlockSpec((_CBLK, C), lambda i: (i, 0)),
            pl.BlockSpec((NC, _CBLK, C), lambda i: (0, i, 0)),
            pl.BlockSpec((1, C), lambda i: (0, 0)),
        ],
        out_specs=pl.BlockSpec((_CBLK, C), lambda i: (i, 0)),
        out_shape=jax.ShapeDtypeStruct((N_PAPER, C), _f32),
    )(p, mw, q_parts, b_out.reshape(1, C))


def _pad_flat(v, fill_ramp):
    n = v.shape[0]
    per = NBUF * NW * CHUNK
    n_pad = -(-n // per) * per
    if fill_ramp:
        pad = DUMP + (jnp.arange(n_pad - n, dtype=jnp.int32)
                      % (NROWS_PAD - N_PAPER))
        return jnp.concatenate([v, pad])
    return jnp.pad(v, (0, n_pad - n))


def kernel(x_paper, x_author, edge_index_writes, edge_index_cites,
           W_paper, b_paper, W_author, b_author, W_out, b_out):
    p = _dense_project(x_paper, W_paper, b_paper, W_out)
    a = _dense_project(x_author, W_author, b_author, W_out)

    ei_w = edge_index_writes.astype(jnp.int32)
    ei_c = edge_index_cites.astype(jnp.int32)
    idx = jnp.stack([
        _pad_flat(ei_w[0], False), _pad_flat(ei_w[1], True),
        _pad_flat(ei_c[0], False), _pad_flat(ei_c[1], True),
    ])
    k = idx.shape[1] // (NW * CHUNK)
    idx = idx.reshape(4, NW, k, CHUNK)

    zeros = jnp.zeros((NROWS_PAD, C), _f32)
    s_col = jnp.arange(NS, dtype=jnp.int32)[:, None]
    x_row = jnp.arange(NQ * CHUNK, dtype=jnp.int32)[None, :]
    ident = jnp.where(
        x_row < ZSTRIPE, s_col * ZSTRIPE + x_row,
        N_PAPER + (s_col * (NQ * CHUNK - ZSTRIPE) + (x_row - ZSTRIPE))
        % (NROWS_PAD - N_PAPER)).reshape(NS, NQ, CHUNK)

    mw_parts = _sc_pass1(a, p, idx, zeros)
    q_parts, mw = _sc_pass2(mw_parts, idx, ident, zeros)
    return _final_combine(p, mw, q_parts, b_out)


# speedup vs baseline: 2.1746x; 1.0090x over previous
import functools

import jax
import jax.numpy as jnp
from jax import lax
from jax.experimental import pallas as pl
from jax.experimental.pallas import tpu as pltpu
from jax.experimental.pallas import tpu_sc as plsc

N_PAPER = 10000
N_AUTHOR = 10000
D = 256
H = 256
C = 64
E = 160000
ALPHA = 0.01

NC = 2
NS = 16
NW = NC * NS
CHUNK = 128
NBUF = 2
DUMP = N_PAPER
NROWS_PAD = 10112
ZSTRIPE = NROWS_PAD // NS
NQ = -(-ZSTRIPE // CHUNK)
NBUF2 = 2

_f32 = jnp.float32


_DENSE_BLK = 1000


def _dense_body(x_ref, w_ref, b_ref, wout_ref, o_ref):
    h = jnp.dot(x_ref[...], w_ref[...], preferred_element_type=_f32)
    h = jnp.maximum(h + b_ref[...], 0.0)
    o_ref[...] = jnp.dot(h, wout_ref[...], preferred_element_type=_f32)


def _dense_project(x, w, b, w_out):
    n = x.shape[0]
    return pl.pallas_call(
        _dense_body,
        grid=(n // _DENSE_BLK,),
        in_specs=[
            pl.BlockSpec((_DENSE_BLK, D), lambda i: (i, 0)),
            pl.BlockSpec((D, H), lambda i: (0, 0)),
            pl.BlockSpec((1, H), lambda i: (0, 0)),
            pl.BlockSpec((H, C), lambda i: (0, 0)),
        ],
        out_specs=pl.BlockSpec((_DENSE_BLK, C), lambda i: (i, 0)),
        out_shape=jax.ShapeDtypeStruct((NROWS_PAD, C), _f32),
    )(x, w, b.reshape(1, H), w_out)


def _edge_pipeline(table, srcv, dstv, k, rows, acc, gsems, ssems, nbuf):
    for b in range(nbuf):
        pltpu.async_copy(table.at[srcv.at[b]], rows.at[b], gsems[b])

    def group(i, carry):
        for b in range(nbuf):
            j = i * nbuf + b
            pltpu.make_async_copy(table.at[srcv.at[j]], rows.at[b],
                                  gsems[b]).wait()
            pltpu.async_copy(rows.at[b], acc.at[dstv.at[j]], ssems[b],
                             add=True)
        for b in range(nbuf):
            j = i * nbuf + b
            pltpu.make_async_copy(rows.at[b], acc.at[dstv.at[j]],
                                  ssems[b]).wait()

            @pl.when(j + nbuf < k)
            def _(b=b, j=j):
                pltpu.async_copy(table.at[srcv.at[j + nbuf]], rows.at[b],
                                 gsems[b])

        return carry

    lax.fori_loop(0, k // nbuf, group, 0)


def _segsum2_body(k, table_a, table_p, idx, zeros, out,
                  sv0, dv0, sv1, dv1, rows, acc, tbl, *sems):
    gsems, ssems = sems[:NBUF], sems[NBUF:]
    c = lax.axis_index("c")
    s = lax.axis_index("s")
    wid = c * NS + s
    stripe = pl.ds(s * ZSTRIPE, ZSTRIPE)
    stage = [(zeros.at[stripe], acc.at[stripe]),
             (table_a.at[stripe], tbl.at[stripe]),
             (idx.at[0, wid], sv0), (idx.at[1, wid], dv0),
             (idx.at[2, wid], sv1), (idx.at[3, wid], dv1)]
    for i, (src, dst) in enumerate(stage):
        pltpu.async_copy(src, dst, sems[i % (2 * NBUF)])
    for i, (src, dst) in enumerate(stage):
        pltpu.make_async_copy(src, dst, sems[i % (2 * NBUF)]).wait()
    plsc.subcore_barrier()
    _edge_pipeline(tbl, sv0, dv0, k, rows, acc, gsems, ssems, NBUF)
    plsc.subcore_barrier()
    pltpu.sync_copy(table_p.at[stripe], tbl.at[stripe])
    plsc.subcore_barrier()
    _edge_pipeline(tbl, sv1, dv1, k, rows, acc, gsems, ssems, NBUF)
    plsc.subcore_barrier()
    pltpu.sync_copy(acc.at[stripe], out.at[c, stripe])


def _segsum_final_body(k, parts, idx, ident, zeros, out, mw_out,
                       sv, dv, idv, rows, acc, tbl, *sems):
    gsems, ssems, wsem = sems[:NBUF2], sems[NBUF2:2 * NBUF2], sems[-1]
    c = lax.axis_index("c")
    s = lax.axis_index("s")
    wid = c * NS + s
    stripe = pl.ds(s * ZSTRIPE, ZSTRIPE)
    stage = [(zeros.at[stripe], acc.at[stripe]),
             (parts.at[0, stripe], tbl.at[stripe]),
             (ident.at[s], idv),
             (idx.at[2, wid], sv), (idx.at[3, wid], dv)]
    for i, (src, dst) in enumerate(stage):
        pltpu.async_copy(src, dst, sems[i % (2 * NBUF2)])
    for i, (src, dst) in enumerate(stage):
        pltpu.make_async_copy(src, dst, sems[i % (2 * NBUF2)]).wait()
    def bounce(q, b):
        n_r = min(CHUNK, ZSTRIPE - q * CHUNK)
        return (parts.at[1, pl.ds(s * ZSTRIPE + q * CHUNK, n_r)],
                rows.at[b, pl.ds(0, n_r)])

    pltpu.async_copy(*bounce(0, 0), gsems[0])
    for q in range(NQ):
        b = q % NBUF2
        pltpu.make_async_copy(*bounce(q, b), gsems[b]).wait()
        if q + 1 < NQ:
            pltpu.async_copy(*bounce(q + 1, (q + 1) % NBUF2),
                             gsems[(q + 1) % NBUF2])
        pltpu.sync_copy(rows.at[b], tbl.at[idv.at[q]], add=True)
    plsc.subcore_barrier()

    @pl.when(c == 0)
    def _():
        pltpu.async_copy(tbl.at[stripe], mw_out.at[stripe], wsem)

    _edge_pipeline(tbl, sv, dv, k, rows, acc, gsems, ssems, NBUF2)

    @pl.when(c == 0)
    def _():
        pltpu.make_async_copy(tbl.at[stripe], mw_out.at[stripe],
                              wsem).wait()

    plsc.subcore_barrier()
    pltpu.sync_copy(acc.at[stripe], out.at[c, stripe])


_MESH = dict(core_axis_name="c", subcore_axis_name="s")


def _sc_pass1(table_a, table_p, idx, zeros):
    k = idx.shape[2]
    scratch = ([pltpu.VMEM((k, CHUNK), jnp.int32)] * 4
               + [pltpu.VMEM((NBUF, CHUNK, C), _f32),
                  pltpu.VMEM_SHARED((NROWS_PAD, C), _f32),
                  pltpu.VMEM_SHARED((NROWS_PAD, C), _f32)]
               + [pltpu.SemaphoreType.DMA] * (2 * NBUF))
    kern = functools.partial(
        pl.kernel,
        out_type=jax.ShapeDtypeStruct((NC, NROWS_PAD, C), _f32),
        mesh=plsc.VectorSubcoreMesh(**_MESH),
        scratch_types=scratch,
        compiler_params=pltpu.CompilerParams(use_tc_tiling_on_sc=False),
    )(functools.partial(_segsum2_body, k))
    return kern(table_a, table_p, idx, zeros)


def _sc_pass2(parts, idx, ident, zeros):
    k = idx.shape[2]
    scratch = ([pltpu.VMEM((k, CHUNK), jnp.int32)] * 2
               + [pltpu.VMEM((NQ, CHUNK), jnp.int32),
                  pltpu.VMEM((NBUF2, CHUNK, C), _f32),
                  pltpu.VMEM_SHARED((NROWS_PAD, C), _f32),
                  pltpu.VMEM_SHARED((NROWS_PAD, C), _f32)]
               + [pltpu.SemaphoreType.DMA] * (2 * NBUF2 + 1))
    kern = functools.partial(
        pl.kernel,
        out_type=(jax.ShapeDtypeStruct((NC, NROWS_PAD, C), _f32),
                  jax.ShapeDtypeStruct((NROWS_PAD, C), _f32)),
        mesh=plsc.VectorSubcoreMesh(**_MESH),
        scratch_types=scratch,
        compiler_params=pltpu.CompilerParams(use_tc_tiling_on_sc=False),
    )(functools.partial(_segsum_final_body, k))
    return kern(parts, idx, ident, zeros)


_CBLK = 1000


def _final_body(p_ref, mw_ref, q_ref, b_ref, o_ref):
    o_ref[...] = ((ALPHA * ALPHA) * p_ref[...] + (2.0 * ALPHA) * mw_ref[...]
                  + q_ref[0] + q_ref[1] + b_ref[0])


def _final_combine(p, mw, q_parts, b_out):
    return pl.pallas_call(
        _final_body,
        grid=(N_PAPER // _CBLK,),
        in_specs=[
            pl.BlockSpec((_CBLK, C), lambda i: (i, 0)),
            pl.BlockSpec((_CBLK, C), lambda i: (i, 0)),
            pl.BlockSpec((NC, _CBLK, C), lambda i: (0, i, 0)),
            pl.BlockSpec((1, C), lambda i: (0, 0)),
        ],
        out_specs=pl.BlockSpec((_CBLK, C), lambda i: (i, 0)),
        out_shape=jax.ShapeDtypeStruct((N_PAPER, C), _f32),
    )(p, mw, q_parts, b_out.reshape(1, C))


def _pad_flat(v, fill_ramp):
    n = v.shape[0]
    per = NBUF * NW * CHUNK
    n_pad = -(-n // per) * per
    if fill_ramp:
        pad = DUMP + (jnp.arange(n_pad - n, dtype=jnp.int32)
                      % (NROWS_PAD - N_PAPER))
        return jnp.concatenate([v, pad])
    return jnp.pad(v, (0, n_pad - n))


def kernel(x_paper, x_author, edge_index_writes, edge_index_cites,
           W_paper, b_paper, W_author, b_author, W_out, b_out):
    p = _dense_project(x_paper, W_paper, b_paper, W_out)
    a = _dense_project(x_author, W_author, b_author, W_out)

    ei_w = edge_index_writes.astype(jnp.int32)
    ei_c = edge_index_cites.astype(jnp.int32)
    idx = jnp.stack([
        _pad_flat(ei_w[0], False), _pad_flat(ei_w[1], True),
        _pad_flat(ei_c[0], False), _pad_flat(ei_c[1], True),
    ])
    k = idx.shape[1] // (NW * CHUNK)
    idx = idx.reshape(4, NW, k, CHUNK)

    zeros = jnp.zeros((NROWS_PAD, C), _f32)
    s_col = jnp.arange(NS, dtype=jnp.int32)[:, None]
    x_row = jnp.arange(NQ * CHUNK, dtype=jnp.int32)[None, :]
    ident = jnp.where(
        x_row < ZSTRIPE, s_col * ZSTRIPE + x_row,
        N_PAPER + (s_col * (NQ * CHUNK - ZSTRIPE) + (x_row - ZSTRIPE))
        % (NROWS_PAD - N_PAPER)).reshape(NS, NQ, CHUNK)

    mw_parts = _sc_pass1(a, p, idx, zeros)
    q_parts, mw = _sc_pass2(mw_parts, idx, ident, zeros)
    return _final_combine(p, mw, q_parts, b_out)
